# baseline (speedup 1.0000x reference)
import jax
import jax.numpy as jnp
from jax.experimental import pallas as pl
from jax.experimental.pallas import tpu as pltpu

_BM = 1024
_SUB = 512


def _body(seq_ref, x_ref, w1_ref, b1_ref, w2_ref, b2_ref,
          w3_ref, b3_ref, out_ref, vacc):
    b = pl.program_id(0)
    j = pl.program_id(1)
    nblk = pl.num_programs(1)
    seq = seq_ref[b]

    @pl.when(j == 0)
    def _init():
        vacc[...] = jnp.zeros_like(vacc)

    @pl.when(j * _BM < seq)
    def _compute():
        acc = jnp.zeros((1, vacc.shape[1]), jnp.float32)
        for s in range(_BM // _SUB):
            x = x_ref[0, s * _SUB:(s + 1) * _SUB, :]
            h = jnp.maximum(
                jnp.dot(x, w1_ref[...], preferred_element_type=jnp.float32)
                + b1_ref[...], 0.0).astype(jnp.bfloat16)
            g = jnp.maximum(
                jnp.dot(h, w2_ref[...], preferred_element_type=jnp.float32)
                + b2_ref[...], 0.0)
            row = (jax.lax.broadcasted_iota(jnp.int32, (_SUB, 1), 0)
                   + j * _BM + s * _SUB)
            gm = jnp.where(row < seq, g, 0.0)
            acc += jnp.sum(gm, axis=0, keepdims=True)
        vacc[...] += acc

    @pl.when(j == nblk - 1)
    def _finish():
        out_ref[b] = (jnp.sum(vacc[...] * w3_ref[...])
                      + b3_ref[0, 0] * seq.astype(jnp.float32))


def kernel(inputs, masks, sequence_lengths, W1, b1, W2, b2, W3, b3):
    del masks
    B, M, F = inputs.shape
    H = W1.shape[1]
    nblk = M // _BM

    def x_map(b, j, seq):
        last = (seq[b] - 1) // _BM
        return (b, jnp.minimum(j, last), 0)

    def w_map(b, j, seq):
        return (0, 0)

    grid_spec = pltpu.PrefetchScalarGridSpec(
        num_scalar_prefetch=1,
        grid=(B, nblk),
        in_specs=[
            pl.BlockSpec((1, _BM, F), x_map),
            pl.BlockSpec((F, H), w_map),
            pl.BlockSpec((1, H), w_map),
            pl.BlockSpec((H, H), w_map),
            pl.BlockSpec((1, H), w_map),
            pl.BlockSpec((1, H), w_map),
            pl.BlockSpec(memory_space=pltpu.SMEM),
        ],
        out_specs=pl.BlockSpec(memory_space=pltpu.SMEM),
        scratch_shapes=[pltpu.VMEM((1, H), jnp.float32)],
    )

    out = pl.pallas_call(
        _body,
        grid_spec=grid_spec,
        out_shape=jax.ShapeDtypeStruct((B,), jnp.float32),
    )(sequence_lengths, inputs.astype(jnp.bfloat16), W1.astype(jnp.bfloat16),
      b1.reshape(1, H), W2.astype(jnp.bfloat16), b2.reshape(1, H),
      W3.reshape(1, H), b3.reshape(1, 1))
    return out


# speedup vs baseline: 1.1287x; 1.1287x over previous
import jax
import jax.numpy as jnp
from jax.experimental import pallas as pl
from jax.experimental.pallas import tpu as pltpu

_BM = 1024
_SUB = 256


def _body(seq_ref, x_ref, w1_ref, b1_ref, w2_ref, b2_ref,
          w3_ref, b3_ref, out_ref, vacc):
    b = pl.program_id(0)
    j = pl.program_id(1)
    nblk = pl.num_programs(1)
    seq = seq_ref[b]

    @pl.when(j == 0)
    def _init():
        vacc[...] = jnp.zeros_like(vacc)

    def chain(s):
        x = x_ref[0, s * _SUB:(s + 1) * _SUB, :]
        h = jnp.maximum(
            jnp.dot(x, w1_ref[...], preferred_element_type=jnp.float32)
            + b1_ref[...], 0.0)
        g = jnp.maximum(
            jnp.dot(h, w2_ref[...], preferred_element_type=jnp.float32)
            + b2_ref[...], 0.0)
        row = (jax.lax.broadcasted_iota(jnp.int32, (_SUB, 1), 0)
               + j * _BM + s * _SUB)
        gm = jnp.where(row < seq, g, 0.0)
        vacc[...] += jnp.sum(gm, axis=0, keepdims=True)

    for s in range(_BM // _SUB):
        pl.when(j * _BM + s * _SUB < seq)(lambda s=s: chain(s))

    @pl.when(j == nblk - 1)
    def _finish():
        out_ref[b] = (jnp.sum(vacc[...] * w3_ref[...])
                      + b3_ref[0, 0] * seq.astype(jnp.float32))


def kernel(inputs, masks, sequence_lengths, W1, b1, W2, b2, W3, b3):
    del masks
    B, M, F = inputs.shape
    H = W1.shape[1]
    nblk = M // _BM

    def x_map(b, j, seq):
        last = (seq[b] - 1) // _BM
        return (b, jnp.minimum(j, last), 0)

    def w_map(b, j, seq):
        return (0, 0)

    grid_spec = pltpu.PrefetchScalarGridSpec(
        num_scalar_prefetch=1,
        grid=(B, nblk),
        in_specs=[
            pl.BlockSpec((1, _BM, F), x_map),
            pl.BlockSpec((F, H), w_map),
            pl.BlockSpec((1, H), w_map),
            pl.BlockSpec((H, H), w_map),
            pl.BlockSpec((1, H), w_map),
            pl.BlockSpec((1, H), w_map),
            pl.BlockSpec(memory_space=pltpu.SMEM),
        ],
        out_specs=pl.BlockSpec(memory_space=pltpu.SMEM),
        scratch_shapes=[pltpu.VMEM((1, H), jnp.float32)],
    )

    out = pl.pallas_call(
        _body,
        grid_spec=grid_spec,
        out_shape=jax.ShapeDtypeStruct((B,), jnp.float32),
    )(sequence_lengths, inputs, W1, b1.reshape(1, H),
      W2, b2.reshape(1, H), W3.reshape(1, H), b3.reshape(1, 1))
    return out


# speedup vs baseline: 1.4393x; 1.2751x over previous
import jax
import jax.numpy as jnp
from jax.experimental import pallas as pl
from jax.experimental.pallas import tpu as pltpu

_BM = 1024
_SUB = 512


def _body(seq_ref, x_ref, w1_ref, b1_ref, w2_ref, b2_ref,
          w3_ref, b3_ref, out_ref, vacc):
    b = pl.program_id(0)
    j = pl.program_id(1)
    nblk = pl.num_programs(1)
    seq = seq_ref[b]

    @pl.when(j == 0)
    def _init():
        vacc[...] = jnp.zeros_like(vacc)

    @pl.when(j * _BM < seq)
    def _compute():
        acc = jnp.zeros((1, vacc.shape[1]), jnp.float32)
        for s in range(_BM // _SUB):
            x = x_ref[0, s * _SUB:(s + 1) * _SUB, :]
            h = jnp.maximum(
                jnp.dot(x, w1_ref[...], preferred_element_type=jnp.float32)
                + b1_ref[...], 0.0)
            g = jnp.maximum(
                jnp.dot(h, w2_ref[...], preferred_element_type=jnp.float32)
                + b2_ref[...], 0.0)
            row = (jax.lax.broadcasted_iota(jnp.int32, (_SUB, 1), 0)
                   + j * _BM + s * _SUB)
            gm = jnp.where(row < seq, g, 0.0)
            acc += jnp.sum(gm, axis=0, keepdims=True)
        vacc[...] += acc

    @pl.when(j == nblk - 1)
    def _finish():
        out_ref[b] = (jnp.sum(vacc[...] * w3_ref[...])
                      + b3_ref[0, 0] * seq.astype(jnp.float32))


def kernel(inputs, masks, sequence_lengths, W1, b1, W2, b2, W3, b3):
    del masks
    B, M, F = inputs.shape
    H = W1.shape[1]
    nblk = M // _BM

    def x_map(b, j, seq):
        last = (seq[b] - 1) // _BM
        return (b, jnp.minimum(j, last), 0)

    def w_map(b, j, seq):
        return (0, 0)

    grid_spec = pltpu.PrefetchScalarGridSpec(
        num_scalar_prefetch=1,
        grid=(B, nblk),
        in_specs=[
            pl.BlockSpec((1, _BM, F), x_map),
            pl.BlockSpec((F, H), w_map),
            pl.BlockSpec((1, H), w_map),
            pl.BlockSpec((H, H), w_map),
            pl.BlockSpec((1, H), w_map),
            pl.BlockSpec((1, H), w_map),
            pl.BlockSpec(memory_space=pltpu.SMEM),
        ],
        out_specs=pl.BlockSpec(memory_space=pltpu.SMEM),
        scratch_shapes=[pltpu.VMEM((1, H), jnp.float32)],
    )

    out = pl.pallas_call(
        _body,
        grid_spec=grid_spec,
        out_shape=jax.ShapeDtypeStruct((B,), jnp.float32),
    )(sequence_lengths, inputs, W1, b1.reshape(1, H),
      W2, b2.reshape(1, H), W3.reshape(1, H), b3.reshape(1, 1))
    return out


# speedup vs baseline: 1.7525x; 1.2176x over previous
import jax
import jax.numpy as jnp
from jax.experimental import pallas as pl
from jax.experimental.pallas import tpu as pltpu

_SUB = 512


def _body(seq_ref, x_ref, w1_ref, b1_ref, w2_ref, b2_ref,
          w3_ref, b3_ref, out_ref, vaccs):
    b = pl.program_id(0)
    seq = seq_ref[b]
    nch = vaccs.shape[0]

    vaccs[...] = jnp.zeros_like(vaccs)

    def chain(s):
        x = x_ref[0, s * _SUB:(s + 1) * _SUB, :]
        h = jnp.maximum(
            jnp.dot(x, w1_ref[...], preferred_element_type=jnp.float32)
            + b1_ref[...], 0.0)
        g = jnp.maximum(
            jnp.dot(h, w2_ref[...], preferred_element_type=jnp.float32)
            + b2_ref[...], 0.0)
        row = jax.lax.broadcasted_iota(jnp.int32, (_SUB, 1), 0) + s * _SUB
        gm = jnp.where(row < seq, g, 0.0)
        vaccs[s:s + 1, :] = jnp.sum(gm, axis=0, keepdims=True)

    chain(0)
    for s in range(1, nch):
        pl.when(s * _SUB < seq)(lambda s=s: chain(s))

    total = jnp.sum(vaccs[...], axis=0, keepdims=True)
    out_ref[b] = (jnp.sum(total * w3_ref[...])
                  + b3_ref[0, 0] * seq.astype(jnp.float32))


def kernel(inputs, masks, sequence_lengths, W1, b1, W2, b2, W3, b3):
    del masks
    B, M, F = inputs.shape
    H = W1.shape[1]
    nch = M // _SUB

    def w_map(b, seq):
        return (0, 0)

    grid_spec = pltpu.PrefetchScalarGridSpec(
        num_scalar_prefetch=1,
        grid=(B,),
        in_specs=[
            pl.BlockSpec((1, M, F), lambda b, seq: (b, 0, 0)),
            pl.BlockSpec((F, H), w_map),
            pl.BlockSpec((1, H), w_map),
            pl.BlockSpec((H, H), w_map),
            pl.BlockSpec((1, H), w_map),
            pl.BlockSpec((1, H), w_map),
            pl.BlockSpec(memory_space=pltpu.SMEM),
        ],
        out_specs=pl.BlockSpec(memory_space=pltpu.SMEM),
        scratch_shapes=[pltpu.VMEM((nch, H), jnp.float32)],
    )

    out = pl.pallas_call(
        _body,
        grid_spec=grid_spec,
        out_shape=jax.ShapeDtypeStruct((B,), jnp.float32),
    )(sequence_lengths, inputs, W1, b1.reshape(1, H),
      W2, b2.reshape(1, H), W3.reshape(1, H), b3.reshape(1, 1))
    return out
